# initial kernel scaffold (unmeasured)
import jax
import jax.numpy as jnp
from jax import lax
from jax.experimental import pallas as pl
from jax.experimental.pallas import tpu as pltpu


def kernel(
    t,
):
    def body(*refs):
        pass

    out_shape = jax.ShapeDtypeStruct(..., jnp.float32)
    return pl.pallas_call(body, out_shape=out_shape)(...)


# baseline (device time: 67783 ns/iter reference)
import jax
import jax.numpy as jnp
from jax import lax
from jax.experimental import pallas as pl
from jax.experimental.pallas import tpu as pltpu

W = 32


def kernel(t):
    m, n = t.shape
    rows = m // W

    def body(x_ref, o_ref, gbuf, send_sems, recv1, recv2):
        me = lax.axis_index("i")

        p1 = []
        for s in range(1, W):
            tgt = lax.rem(me + s, W)
            desc = pltpu.make_async_remote_copy(
                src_ref=x_ref.at[pl.ds(tgt * rows, rows), :],
                dst_ref=gbuf.at[me],
                send_sem=send_sems.at[s],
                recv_sem=recv1.at[me],
                device_id=(tgt,),
                device_id_type=pl.DeviceIdType.MESH,
            )
            desc.start()
            p1.append(desc)

        gbuf[pl.ds(me, 1)] = x_ref[pl.ds(me * rows, rows), :].reshape(1, rows, n)

        for s in range(1, W):
            src_dev = lax.rem(me + s, W)
            pltpu.make_async_remote_copy(
                src_ref=gbuf.at[src_dev],
                dst_ref=gbuf.at[src_dev],
                send_sem=send_sems.at[s],
                recv_sem=recv1.at[src_dev],
                device_id=(src_dev,),
                device_id_type=pl.DeviceIdType.MESH,
            ).wait_recv()

        acc = jnp.sum(gbuf[...], axis=0)
        r = jnp.maximum(acc, 0.0)
        res = jnp.tanh(acc) * acc * acc + r * r * r
        o_ref[pl.ds(me * rows, rows), :] = res

        for desc in p1:
            desc.wait_send()

        p2 = []
        for s in range(1, W):
            tgt = lax.rem(me + s, W)
            desc = pltpu.make_async_remote_copy(
                src_ref=o_ref.at[pl.ds(me * rows, rows), :],
                dst_ref=o_ref.at[pl.ds(me * rows, rows), :],
                send_sem=send_sems.at[s],
                recv_sem=recv2.at[me],
                device_id=(tgt,),
                device_id_type=pl.DeviceIdType.MESH,
            )
            desc.start()
            p2.append(desc)

        for s in range(1, W):
            src_dev = lax.rem(me + s, W)
            pltpu.make_async_remote_copy(
                src_ref=o_ref.at[pl.ds(src_dev * rows, rows), :],
                dst_ref=o_ref.at[pl.ds(src_dev * rows, rows), :],
                send_sem=send_sems.at[s],
                recv_sem=recv2.at[src_dev],
                device_id=(src_dev,),
                device_id_type=pl.DeviceIdType.MESH,
            ).wait_recv()

        for desc in p2:
            desc.wait_send()

    return pl.pallas_call(
        body,
        out_shape=jax.ShapeDtypeStruct((m, n), jnp.float32),
        in_specs=[pl.BlockSpec(memory_space=pltpu.VMEM)],
        out_specs=pl.BlockSpec(memory_space=pltpu.VMEM),
        scratch_shapes=[
            pltpu.VMEM((W, rows, n), jnp.float32),
            pltpu.SemaphoreType.DMA((W,)),
            pltpu.SemaphoreType.DMA((W,)),
            pltpu.SemaphoreType.DMA((W,)),
        ],
    )(t)


# device time: 59435 ns/iter; 1.1405x vs baseline; 1.1405x over previous
import jax
import jax.numpy as jnp
from jax import lax
from jax.experimental import pallas as pl
from jax.experimental.pallas import tpu as pltpu

W = 32


def kernel(t):
    m, n = t.shape
    rows = m // W

    def body(x_ref, o_ref, gbuf, send_sems, recv1, recv2):
        me = lax.axis_index("i")

        barrier_sem = pltpu.get_barrier_semaphore()
        for s in range(1, W):
            pl.semaphore_signal(
                barrier_sem, inc=1,
                device_id=(lax.rem(me + s, W),),
                device_id_type=pl.DeviceIdType.MESH,
            )
        pl.semaphore_wait(barrier_sem, W - 1)

        p1 = []
        for s in range(1, W):
            tgt = lax.rem(me + s, W)
            desc = pltpu.make_async_remote_copy(
                src_ref=x_ref.at[pl.ds(tgt * rows, rows), :],
                dst_ref=gbuf.at[me],
                send_sem=send_sems.at[s],
                recv_sem=recv1.at[me],
                device_id=(tgt,),
                device_id_type=pl.DeviceIdType.MESH,
            )
            desc.start()
            p1.append(desc)

        acc = x_ref[pl.ds(me * rows, rows), :]
        for s in range(1, W):
            src_dev = lax.rem(me + s, W)
            pltpu.make_async_remote_copy(
                src_ref=gbuf.at[src_dev],
                dst_ref=gbuf.at[src_dev],
                send_sem=send_sems.at[s],
                recv_sem=recv1.at[src_dev],
                device_id=(src_dev,),
                device_id_type=pl.DeviceIdType.MESH,
            ).wait_recv()
            acc = acc + gbuf[src_dev]
        r = jnp.maximum(acc, 0.0)
        res = jnp.tanh(acc) * acc * acc + r * r * r
        o_ref[pl.ds(me * rows, rows), :] = res

        for desc in p1:
            desc.wait_send()

        p2 = []
        for s in range(1, W):
            tgt = lax.rem(me + s, W)
            desc = pltpu.make_async_remote_copy(
                src_ref=o_ref.at[pl.ds(me * rows, rows), :],
                dst_ref=o_ref.at[pl.ds(me * rows, rows), :],
                send_sem=send_sems.at[s],
                recv_sem=recv2.at[me],
                device_id=(tgt,),
                device_id_type=pl.DeviceIdType.MESH,
            )
            desc.start()
            p2.append(desc)

        for s in range(1, W):
            src_dev = lax.rem(me + s, W)
            pltpu.make_async_remote_copy(
                src_ref=o_ref.at[pl.ds(src_dev * rows, rows), :],
                dst_ref=o_ref.at[pl.ds(src_dev * rows, rows), :],
                send_sem=send_sems.at[s],
                recv_sem=recv2.at[src_dev],
                device_id=(src_dev,),
                device_id_type=pl.DeviceIdType.MESH,
            ).wait_recv()

        for desc in p2:
            desc.wait_send()

    return pl.pallas_call(
        body,
        out_shape=jax.ShapeDtypeStruct((m, n), jnp.float32),
        in_specs=[pl.BlockSpec(memory_space=pltpu.VMEM)],
        out_specs=pl.BlockSpec(memory_space=pltpu.VMEM),
        scratch_shapes=[
            pltpu.VMEM((W, rows, n), jnp.float32),
            pltpu.SemaphoreType.DMA((W,)),
            pltpu.SemaphoreType.DMA((W,)),
            pltpu.SemaphoreType.DMA((W,)),
        ],
        compiler_params=pltpu.CompilerParams(collective_id=0),
    )(t)


# device time: 38703 ns/iter; 1.7514x vs baseline; 1.5357x over previous
import jax
import jax.numpy as jnp
from jax import lax
from jax.experimental import pallas as pl
from jax.experimental.pallas import tpu as pltpu

W = 32


def kernel(t):
    m, n = t.shape
    rows = m // W

    def body(x_ref, o_ref, sbuf, gbuf, bbuf, obuf, send_sems, recv1, recv2):
        me = lax.axis_index("i")

        sbuf[...] = x_ref[...].astype(jnp.bfloat16)

        barrier_sem = pltpu.get_barrier_semaphore()
        for s in range(1, W):
            pl.semaphore_signal(
                barrier_sem, inc=1,
                device_id=(lax.rem(me + s, W),),
                device_id_type=pl.DeviceIdType.MESH,
            )
        pl.semaphore_wait(barrier_sem, W - 1)

        p1 = []
        for s in range(1, W):
            tgt = lax.rem(me + s, W)
            desc = pltpu.make_async_remote_copy(
                src_ref=sbuf.at[pl.ds(tgt * rows, rows), :],
                dst_ref=gbuf.at[me],
                send_sem=send_sems.at[s],
                recv_sem=recv1.at[me],
                device_id=(tgt,),
                device_id_type=pl.DeviceIdType.MESH,
            )
            desc.start()
            p1.append(desc)

        acc = x_ref[pl.ds(me * rows, rows), :]
        for s in range(1, W):
            src_dev = lax.rem(me + s, W)
            pltpu.make_async_remote_copy(
                src_ref=gbuf.at[src_dev],
                dst_ref=gbuf.at[src_dev],
                send_sem=send_sems.at[s],
                recv_sem=recv1.at[src_dev],
                device_id=(src_dev,),
                device_id_type=pl.DeviceIdType.MESH,
            ).wait_recv()
            acc = acc + gbuf[src_dev].astype(jnp.float32)

        r = jnp.maximum(acc, 0.0)
        res = jnp.tanh(acc) * acc * acc + r * r * r
        o_ref[pl.ds(me * rows, rows), :] = res
        bbuf[...] = res.astype(jnp.bfloat16)

        for desc in p1:
            desc.wait_send()

        p2 = []
        for s in range(1, W):
            tgt = lax.rem(me + s, W)
            desc = pltpu.make_async_remote_copy(
                src_ref=bbuf,
                dst_ref=obuf.at[me],
                send_sem=send_sems.at[s],
                recv_sem=recv2.at[me],
                device_id=(tgt,),
                device_id_type=pl.DeviceIdType.MESH,
            )
            desc.start()
            p2.append(desc)

        for s in range(1, W):
            src_dev = lax.rem(me + s, W)
            pltpu.make_async_remote_copy(
                src_ref=obuf.at[src_dev],
                dst_ref=obuf.at[src_dev],
                send_sem=send_sems.at[s],
                recv_sem=recv2.at[src_dev],
                device_id=(src_dev,),
                device_id_type=pl.DeviceIdType.MESH,
            ).wait_recv()
            o_ref[pl.ds(src_dev * rows, rows), :] = obuf[src_dev].astype(
                jnp.float32
            )

        for desc in p2:
            desc.wait_send()

    return pl.pallas_call(
        body,
        out_shape=jax.ShapeDtypeStruct((m, n), jnp.float32),
        in_specs=[pl.BlockSpec(memory_space=pltpu.VMEM)],
        out_specs=pl.BlockSpec(memory_space=pltpu.VMEM),
        scratch_shapes=[
            pltpu.VMEM((m, n), jnp.bfloat16),
            pltpu.VMEM((W, rows, n), jnp.bfloat16),
            pltpu.VMEM((rows, n), jnp.bfloat16),
            pltpu.VMEM((W, rows, n), jnp.bfloat16),
            pltpu.SemaphoreType.DMA((W,)),
            pltpu.SemaphoreType.DMA((W,)),
            pltpu.SemaphoreType.DMA((W,)),
        ],
        compiler_params=pltpu.CompilerParams(collective_id=0),
    )(t)


# device time: 38447 ns/iter; 1.7630x vs baseline; 1.0067x over previous
import jax
import jax.numpy as jnp
from jax import lax
from jax.experimental import pallas as pl
from jax.experimental.pallas import tpu as pltpu

W = 32


def kernel(t):
    m, n = t.shape
    rows = m // W

    def body(x_ref, o_ref, sbuf, gbuf, bbuf, obuf, send_sems, recv1, recv2):
        me = lax.axis_index("i")

        barrier_sem = pltpu.get_barrier_semaphore()
        for s in range(1, W):
            pl.semaphore_signal(
                barrier_sem, inc=1,
                device_id=(lax.rem(me + s, W),),
                device_id_type=pl.DeviceIdType.MESH,
            )

        sbuf[...] = x_ref[...].astype(jnp.bfloat16)

        pl.semaphore_wait(barrier_sem, W - 1)

        p1 = []
        for s in range(1, W):
            tgt = lax.rem(me + s, W)
            desc = pltpu.make_async_remote_copy(
                src_ref=sbuf.at[pl.ds(tgt * rows, rows), :],
                dst_ref=gbuf.at[me],
                send_sem=send_sems.at[s],
                recv_sem=recv1.at[me],
                device_id=(tgt,),
                device_id_type=pl.DeviceIdType.MESH,
            )
            desc.start()
            p1.append(desc)

        acc = x_ref[pl.ds(me * rows, rows), :]
        for s in range(1, W):
            src_dev = lax.rem(me + s, W)
            pltpu.make_async_remote_copy(
                src_ref=gbuf.at[src_dev],
                dst_ref=gbuf.at[src_dev],
                send_sem=send_sems.at[s],
                recv_sem=recv1.at[src_dev],
                device_id=(src_dev,),
                device_id_type=pl.DeviceIdType.MESH,
            ).wait_recv()
            acc = acc + gbuf[src_dev].astype(jnp.float32)

        r = jnp.maximum(acc, 0.0)
        res = jnp.tanh(acc) * acc * acc + r * r * r
        o_ref[pl.ds(me * rows, rows), :] = res
        bbuf[...] = res.astype(jnp.bfloat16)

        for desc in p1:
            desc.wait_send()

        p2 = []
        for s in range(1, W):
            tgt = lax.rem(me + s, W)
            desc = pltpu.make_async_remote_copy(
                src_ref=bbuf,
                dst_ref=obuf.at[me],
                send_sem=send_sems.at[s],
                recv_sem=recv2.at[me],
                device_id=(tgt,),
                device_id_type=pl.DeviceIdType.MESH,
            )
            desc.start()
            p2.append(desc)

        for s in range(1, W):
            src_dev = lax.rem(me + s, W)
            pltpu.make_async_remote_copy(
                src_ref=obuf.at[src_dev],
                dst_ref=obuf.at[src_dev],
                send_sem=send_sems.at[s],
                recv_sem=recv2.at[src_dev],
                device_id=(src_dev,),
                device_id_type=pl.DeviceIdType.MESH,
            ).wait_recv()
            o_ref[pl.ds(src_dev * rows, rows), :] = obuf[src_dev].astype(
                jnp.float32
            )

        for desc in p2:
            desc.wait_send()

    return pl.pallas_call(
        body,
        out_shape=jax.ShapeDtypeStruct((m, n), jnp.float32),
        in_specs=[pl.BlockSpec(memory_space=pltpu.VMEM)],
        out_specs=pl.BlockSpec(memory_space=pltpu.VMEM),
        scratch_shapes=[
            pltpu.VMEM((m, n), jnp.bfloat16),
            pltpu.VMEM((W, rows, n), jnp.bfloat16),
            pltpu.VMEM((rows, n), jnp.bfloat16),
            pltpu.VMEM((W, rows, n), jnp.bfloat16),
            pltpu.SemaphoreType.DMA((W,)),
            pltpu.SemaphoreType.DMA((W,)),
            pltpu.SemaphoreType.DMA((W,)),
        ],
        compiler_params=pltpu.CompilerParams(collective_id=0),
    )(t)


# device time: 37180 ns/iter; 1.8231x vs baseline; 1.0341x over previous
import jax
import jax.numpy as jnp
from jax import lax
from jax.experimental import pallas as pl
from jax.experimental.pallas import tpu as pltpu

W = 32


def kernel(t):
    m, n = t.shape
    rows = m // W

    def body(x_ref, o_ref, sbuf, gbuf, bbuf, obuf, send_sems, recv1, recv2,
             ready):
        me = lax.axis_index("i")

        barrier_sem = pltpu.get_barrier_semaphore()
        pl.semaphore_signal(
            barrier_sem, inc=1, device_id=(me,),
            device_id_type=pl.DeviceIdType.MESH,
        )
        pl.semaphore_wait(barrier_sem, 1)

        for s in range(1, W):
            pl.semaphore_signal(
                ready.at[me], inc=1,
                device_id=(lax.rem(me + s, W),),
                device_id_type=pl.DeviceIdType.MESH,
            )

        sbuf[...] = x_ref[...].astype(jnp.bfloat16)

        order = []
        for k in range(1, W // 2 + 1):
            order.append(k)
            if k != W - k:
                order.append(W - k)
        p1 = []
        for s in order:
            tgt = lax.rem(me + s, W)
            pl.semaphore_wait(ready.at[tgt], 1)
            desc = pltpu.make_async_remote_copy(
                src_ref=sbuf.at[pl.ds(tgt * rows, rows), :],
                dst_ref=gbuf.at[me],
                send_sem=send_sems.at[s],
                recv_sem=recv1.at[me],
                device_id=(tgt,),
                device_id_type=pl.DeviceIdType.MESH,
            )
            desc.start()
            p1.append(desc)

        acc = x_ref[pl.ds(me * rows, rows), :]
        for s in range(1, W):
            src_dev = lax.rem(me + s, W)
            pltpu.make_async_remote_copy(
                src_ref=gbuf.at[src_dev],
                dst_ref=gbuf.at[src_dev],
                send_sem=send_sems.at[s],
                recv_sem=recv1.at[src_dev],
                device_id=(src_dev,),
                device_id_type=pl.DeviceIdType.MESH,
            ).wait_recv()
            acc = acc + gbuf[src_dev].astype(jnp.float32)

        r = jnp.maximum(acc, 0.0)
        res = jnp.tanh(acc) * acc * acc + r * r * r
        o_ref[pl.ds(me * rows, rows), :] = res
        bbuf[...] = res.astype(jnp.bfloat16)

        for desc in p1:
            desc.wait_send()

        p2 = []
        for s in range(1, W):
            tgt = lax.rem(me + s, W)
            desc = pltpu.make_async_remote_copy(
                src_ref=bbuf,
                dst_ref=obuf.at[me],
                send_sem=send_sems.at[s],
                recv_sem=recv2.at[me],
                device_id=(tgt,),
                device_id_type=pl.DeviceIdType.MESH,
            )
            desc.start()
            p2.append(desc)

        for s in range(1, W):
            src_dev = lax.rem(me + s, W)
            pltpu.make_async_remote_copy(
                src_ref=obuf.at[src_dev],
                dst_ref=obuf.at[src_dev],
                send_sem=send_sems.at[s],
                recv_sem=recv2.at[src_dev],
                device_id=(src_dev,),
                device_id_type=pl.DeviceIdType.MESH,
            ).wait_recv()
            o_ref[pl.ds(src_dev * rows, rows), :] = obuf[src_dev].astype(
                jnp.float32
            )

        for desc in p2:
            desc.wait_send()

    return pl.pallas_call(
        body,
        out_shape=jax.ShapeDtypeStruct((m, n), jnp.float32),
        in_specs=[pl.BlockSpec(memory_space=pltpu.VMEM)],
        out_specs=pl.BlockSpec(memory_space=pltpu.VMEM),
        scratch_shapes=[
            pltpu.VMEM((m, n), jnp.bfloat16),
            pltpu.VMEM((W, rows, n), jnp.bfloat16),
            pltpu.VMEM((rows, n), jnp.bfloat16),
            pltpu.VMEM((W, rows, n), jnp.bfloat16),
            pltpu.SemaphoreType.DMA((W,)),
            pltpu.SemaphoreType.DMA((W,)),
            pltpu.SemaphoreType.DMA((W,)),
            pltpu.SemaphoreType.REGULAR((W,)),
        ],
        compiler_params=pltpu.CompilerParams(collective_id=0),
    )(t)
